# baseline (speedup 1.0000x reference)
import functools

import jax
import jax.numpy as jnp
from jax import lax
from jax.experimental import pallas as pl
from jax.experimental.pallas import tpu as pltpu
from jax.experimental.pallas import tpu_sc as plsc

_N = 100000
_D = 32
_E = 1600000
_EPAD = 1638400
_G = 256
_SG = 8
_CH = _G * _SG
_NCHUNK = _EPAD // (16 * _CH)
_ER = _EPAD // _G
_HALF = _N // 2
_SPREAD = 16383
_ACC_ROWS = 50048
_ZA = 3128
_FLA = 3128
_FLB = _HALF - 15 * _FLA
_CAT = 0.55


def _make_spmm():
    mesh = plsc.VectorSubcoreMesh(core_axis_name="c", subcore_axis_name="s")

    def body(dst_hbm, src_hbm, val_hbm, x_hbm, z_hbm,
             idx_d, idx_s, vals_v, rows_a, rows_b, acc,
             gsem_a, gsem_b, ssem_a, ssem_b):
        c = lax.axis_index("c")
        s = lax.axis_index("s")
        base = c * _HALF

        def _zrow(e, carry):
            rows_a[e, 0:16] = jnp.zeros((16,), jnp.float32)
            rows_a[e, 16:32] = jnp.zeros((16,), jnp.float32)
            return carry
        lax.fori_loop(0, _G, _zrow, 0)

        def _zcp(b, carry):
            pltpu.sync_copy(rows_a.at[pl.ds(0, 128)],
                            acc.at[pl.ds(s * _ZA + b * 128, 128)])
            return carry
        lax.fori_loop(0, 24, _zcp, 0)
        pltpu.sync_copy(rows_a.at[pl.ds(0, 56)],
                        acc.at[pl.ds(s * _ZA + 24 * 128, 56)])
        plsc.subcore_barrier()

        def gather_start(j, rows_x, sem):
            pltpu.async_copy(x_hbm.at[idx_s.at[j]], rows_x, sem)

        def gather_wait(j, rows_x, sem):
            pltpu.make_async_copy(x_hbm.at[idx_s.at[j]], rows_x, sem).wait()

        def scat_start(j, rows_x, sem):
            pltpu.async_copy(rows_x, acc.at[idx_d.at[j]], sem, add=True)

        def scat_wait(j, rows_x, sem):
            pltpu.make_async_copy(rows_x, acc.at[idx_d.at[j]], sem).wait()

        def scale(j, rows_x):
            def sk(k, carry):
                vv = vals_v[j, pl.ds(k * 16, 16)]
                for i in range(16):
                    e = k * 16 + i
                    rows_x[e, 0:16] = rows_x[e, 0:16] * vv[i]
                    rows_x[e, 16:32] = rows_x[e, 16:32] * vv[i]
                return carry
            lax.fori_loop(0, _G // 16, sk, 0)

        def chunk(ch, carry):
            rb = (s * _NCHUNK + ch) * _SG
            pltpu.sync_copy(dst_hbm.at[pl.ds(rb, _SG)], idx_d)
            pltpu.sync_copy(src_hbm.at[pl.ds(rb, _SG)], idx_s)
            pltpu.sync_copy(val_hbm.at[pl.ds(rb, _SG)], vals_v)

            def bulkprep(k, carry2):
                r = k // 16
                sl = pl.ds((k % 16) * 16, 16)
                d = idx_d[r, sl]
                l = d - base
                inb = (l >= 0) & (l < _HALF)
                idx_d[r, sl] = jnp.where(inb, l, lax.bitwise_and(d, _SPREAD))
                vals_v[r, sl] = jnp.where(inb, vals_v[r, sl],
                                          jnp.zeros((16,), jnp.float32))
                return carry2
            lax.fori_loop(0, _CH // 16, bulkprep, 0, unroll=8)

            gather_start(0, rows_a, gsem_a)

            def pair(jj, carry2):
                j0 = 2 * jj

                @pl.when(jj > 0)
                def _():
                    scat_wait(j0 - 1, rows_b, ssem_b)
                gather_start(j0 + 1, rows_b, gsem_b)
                gather_wait(j0, rows_a, gsem_a)
                scale(j0, rows_a)
                scat_start(j0, rows_a, ssem_a)

                scat_wait(j0, rows_a, ssem_a)

                @pl.when(j0 + 2 < _SG)
                def _():
                    gather_start(j0 + 2, rows_a, gsem_a)
                gather_wait(j0 + 1, rows_b, gsem_b)
                scale(j0 + 1, rows_b)
                scat_start(j0 + 1, rows_b, ssem_b)
                return carry2
            lax.fori_loop(0, _SG // 2, pair, 0)
            scat_wait(_SG - 1, rows_b, ssem_b)
            return carry
        lax.fori_loop(0, _NCHUNK, chunk, 0)

        plsc.subcore_barrier()

        @pl.when(s < 15)
        def _flush_main():
            pltpu.sync_copy(acc.at[pl.ds(s * _FLA, _FLA)],
                            z_hbm.at[pl.ds(base + s * _FLA, _FLA)])

        @pl.when(s == 15)
        def _flush_tail():
            pltpu.sync_copy(acc.at[pl.ds(15 * _FLA, _FLB)],
                            z_hbm.at[pl.ds(base + 15 * _FLA, _FLB)])

    return pl.kernel(
        body,
        out_type=jax.ShapeDtypeStruct((_N, _D), jnp.float32),
        mesh=mesh,
        compiler_params=pltpu.CompilerParams(use_tc_tiling_on_sc=False),
        scratch_types=[
            pltpu.VMEM((_SG, _G), jnp.int32),
            pltpu.VMEM((_SG, _G), jnp.int32),
            pltpu.VMEM((_SG, _G), jnp.float32),
            pltpu.VMEM((_G, _D), jnp.float32),
            pltpu.VMEM((_G, _D), jnp.float32),
            pltpu.VMEM_SHARED((_ACC_ROWS, _D), jnp.float32),
            pltpu.SemaphoreType.DMA,
            pltpu.SemaphoreType.DMA,
            pltpu.SemaphoreType.DMA,
            pltpu.SemaphoreType.DMA,
        ],
    )


_spmm = _make_spmm()


def _nrm(x):
    n = jnp.sqrt(jnp.sum(x * x, axis=1, keepdims=True))
    return x / jnp.maximum(n, 1e-12)


def _combine_body(ue0, zu1, zu2, giu, gtu, ie0, zi1, zi2, gii, gti, uo, io):
    uo[...] = (ue0[...] + zu1[...] + zu2[...]) * (1.0 / 3.0) \
        + _CAT * (_nrm(giu[...]) + _nrm(gtu[...]))
    io[...] = (ie0[...] + zi1[...] + zi2[...]) * (1.0 / 3.0) \
        + _CAT * (_nrm(gii[...]) + _nrm(gti[...]))


_BLK = 2000


def _combine(ue0, zu1, zu2, giu, gtu, ie0, zi1, zi2, gii, gti):
    spec = pl.BlockSpec((_BLK, _D), lambda i: (i, 0))
    return pl.pallas_call(
        _combine_body,
        grid=(_N // _BLK,),
        in_specs=[spec] * 10,
        out_specs=[spec, spec],
        out_shape=[jax.ShapeDtypeStruct((_N, _D), jnp.float32)] * 2,
    )(ue0, zu1, zu2, giu, gtu, ie0, zi1, zi2, gii, gti)


def kernel(adj_indices, adj_values, image_item_embeds, text_item_embeds,
           image_user_embeds, text_user_embeds, user_emb, item_emb,
           user_emb_pre, item_emb_pre):
    rows = adj_indices[0].astype(jnp.int32)
    cols = adj_indices[1].astype(jnp.int32)
    vals = adj_values.astype(jnp.float32)

    npad = _EPAD - _E
    pad_idx = (jnp.arange(npad, dtype=jnp.int32) * 7) % _N
    rows_p = jnp.concatenate([rows, pad_idx]).reshape(_ER, _G)
    cols_p = jnp.concatenate([cols, pad_idx]).reshape(_ER, _G)
    vals_p = jnp.concatenate(
        [vals, jnp.zeros((npad,), jnp.float32)]).reshape(_ER, _G)

    ue0 = user_emb_pre + user_emb
    ie0 = item_emb_pre + item_emb

    z_u1 = _spmm(rows_p, cols_p, vals_p, ie0)
    z_i1 = _spmm(cols_p, rows_p, vals_p, ue0)
    z_u2 = _spmm(rows_p, cols_p, vals_p, z_i1)
    z_i2 = _spmm(cols_p, rows_p, vals_p, z_u1)

    return _combine(ue0, z_u1, z_u2, image_user_embeds, text_user_embeds,
                    ie0, z_i1, z_i2, image_item_embeds, text_item_embeds)


# speedup vs baseline: 1.1435x; 1.1435x over previous
import functools

import jax
import jax.numpy as jnp
from jax import lax
from jax.experimental import pallas as pl
from jax.experimental.pallas import tpu as pltpu
from jax.experimental.pallas import tpu_sc as plsc

_N = 100000
_D = 32
_E = 1600000
_EPAD = 1638400
_G = 128
_SG = 16
_CH = _G * _SG
_NCHUNK = _EPAD // (16 * _CH)
_HALF = _N // 2
_SPREAD = 16383
_ACC_ROWS = 51200
_ZB = _ACC_ROWS // (16 * _G)
_FLA = 3128
_FLB = _HALF - 15 * _FLA
_CAT = 0.55


def _make_spmm():
    mesh = plsc.VectorSubcoreMesh(core_axis_name="c", subcore_axis_name="s")

    def body(dst_hbm, src_hbm, val_hbm, x_hbm, z_hbm,
             idx_d, idx_s, vals_v, vals_m, lidx_ch,
             rows0, rows1, rows2, rows3, acc,
             g0, g1, g2, g3, s0, s1, s2, s3):
        c = lax.axis_index("c")
        s = lax.axis_index("s")
        base = c * _HALF
        rows = (rows0, rows1, rows2, rows3)
        gsem = (g0, g1, g2, g3)
        ssem = (s0, s1, s2, s3)

        def _zrow(e, carry):
            rows0[e, 0:16] = jnp.zeros((16,), jnp.float32)
            rows0[e, 16:32] = jnp.zeros((16,), jnp.float32)
            return carry
        lax.fori_loop(0, _G, _zrow, 0)

        def _zcp(b, carry):
            pltpu.sync_copy(rows0, acc.at[pl.ds(s * (_ZB * _G) + b * _G, _G)])
            return carry
        lax.fori_loop(0, _ZB, _zcp, 0)
        plsc.subcore_barrier()

        def gather_start(j, rows_x, sem):
            pltpu.async_copy(x_hbm.at[idx_s.at[pl.ds(j * _G, _G)]], rows_x, sem)

        def gather_wait(j, rows_x, sem):
            pltpu.make_async_copy(
                x_hbm.at[idx_s.at[pl.ds(j * _G, _G)]], rows_x, sem).wait()

        def scat_start(j, rows_x, sem):
            pltpu.async_copy(rows_x, acc.at[lidx_ch.at[j]], sem, add=True)

        def scat_wait(j, rows_x, sem):
            pltpu.make_async_copy(rows_x, acc.at[lidx_ch.at[j]], sem).wait()

        def scale(j, rows_x):
            def sk(k, carry):
                vv = vals_m[pl.ds(j * _G + k * 16, 16)]
                for i in range(16):
                    e = k * 16 + i
                    rows_x[e, 0:16] = rows_x[e, 0:16] * vv[i]
                    rows_x[e, 16:32] = rows_x[e, 16:32] * vv[i]
                return carry
            lax.fori_loop(0, _G // 16, sk, 0)

        def chunk(ch, carry):
            ebase = (s * _NCHUNK + ch) * _CH
            pltpu.sync_copy(dst_hbm.at[pl.ds(ebase, _CH)], idx_d)
            pltpu.sync_copy(src_hbm.at[pl.ds(ebase, _CH)], idx_s)
            pltpu.sync_copy(val_hbm.at[pl.ds(ebase, _CH)], vals_v)

            def bulkprep(k, carry2):
                sl = pl.ds(k * 16, 16)
                d = idx_d[sl]
                l = d - base
                inb = (l >= 0) & (l < _HALF)
                spread = lax.bitwise_and(d, _SPREAD)
                lidx_ch[k // 8, pl.ds((k % 8) * 16, 16)] = jnp.where(inb, l, spread)
                vals_m[sl] = jnp.where(inb, vals_v[sl],
                                       jnp.zeros((16,), jnp.float32))
                return carry2
            lax.fori_loop(0, _CH // 16, bulkprep, 0, unroll=8)

            gather_start(0, rows0, gsem[0])
            gather_start(1, rows1, gsem[1])

            def quad(jj, carry2):
                j0 = 4 * jj
                for x in range(4):
                    j = j0 + x
                    p = (x + 2) % 4

                    @pl.when(j >= 2)
                    def _(j=j, p=p):
                        scat_wait(j - 2, rows[p], ssem[p])

                    @pl.when(j + 2 < _SG)
                    def _(j=j, p=p):
                        gather_start(j + 2, rows[p], gsem[p])
                    gather_wait(j, rows[x], gsem[x])
                    scale(j, rows[x])
                    scat_start(j, rows[x], ssem[x])
                return carry2
            lax.fori_loop(0, _SG // 4, quad, 0)
            scat_wait(_SG - 2, rows[(_SG - 2) % 4], ssem[(_SG - 2) % 4])
            scat_wait(_SG - 1, rows[(_SG - 1) % 4], ssem[(_SG - 1) % 4])
            return carry
        lax.fori_loop(0, _NCHUNK, chunk, 0)

        plsc.subcore_barrier()

        @pl.when(s < 15)
        def _flush_main():
            pltpu.sync_copy(acc.at[pl.ds(s * _FLA, _FLA)],
                            z_hbm.at[pl.ds(base + s * _FLA, _FLA)])

        @pl.when(s == 15)
        def _flush_tail():
            pltpu.sync_copy(acc.at[pl.ds(15 * _FLA, _FLB)],
                            z_hbm.at[pl.ds(base + 15 * _FLA, _FLB)])

    return pl.kernel(
        body,
        out_type=jax.ShapeDtypeStruct((_N, _D), jnp.float32),
        mesh=mesh,
        compiler_params=pltpu.CompilerParams(use_tc_tiling_on_sc=False),
        scratch_types=[
            pltpu.VMEM((_CH,), jnp.int32),
            pltpu.VMEM((_CH,), jnp.int32),
            pltpu.VMEM((_CH,), jnp.float32),
            pltpu.VMEM((_CH,), jnp.float32),
            pltpu.VMEM((_SG, _G), jnp.int32),
            pltpu.VMEM((_G, _D), jnp.float32),
            pltpu.VMEM((_G, _D), jnp.float32),
            pltpu.VMEM((_G, _D), jnp.float32),
            pltpu.VMEM((_G, _D), jnp.float32),
            pltpu.VMEM_SHARED((_ACC_ROWS, _D), jnp.float32),
            pltpu.SemaphoreType.DMA,
            pltpu.SemaphoreType.DMA,
            pltpu.SemaphoreType.DMA,
            pltpu.SemaphoreType.DMA,
            pltpu.SemaphoreType.DMA,
            pltpu.SemaphoreType.DMA,
            pltpu.SemaphoreType.DMA,
            pltpu.SemaphoreType.DMA,
        ],
    )


_spmm = _make_spmm()


def _nrm(x):
    n = jnp.sqrt(jnp.sum(x * x, axis=1, keepdims=True))
    return x / jnp.maximum(n, 1e-12)


def _combine_body(e0, z1, z2, gi, gt, out):
    out[...] = (e0[...] + z1[...] + z2[...]) * (1.0 / 3.0) \
        + _CAT * (_nrm(gi[...]) + _nrm(gt[...]))


_BLK = 2000


def _combine(e0, z1, z2, gi, gt):
    spec = pl.BlockSpec((_BLK, _D), lambda i: (i, 0))
    return pl.pallas_call(
        _combine_body,
        grid=(_N // _BLK,),
        in_specs=[spec] * 5,
        out_specs=spec,
        out_shape=jax.ShapeDtypeStruct((_N, _D), jnp.float32),
    )(e0, z1, z2, gi, gt)


def kernel(adj_indices, adj_values, image_item_embeds, text_item_embeds,
           image_user_embeds, text_user_embeds, user_emb, item_emb,
           user_emb_pre, item_emb_pre):
    rows = adj_indices[0].astype(jnp.int32)
    cols = adj_indices[1].astype(jnp.int32)
    vals = adj_values.astype(jnp.float32)

    npad = _EPAD - _E
    pad_idx = (jnp.arange(npad, dtype=jnp.int32) * 7) % _N
    rows_p = jnp.concatenate([rows, pad_idx])
    cols_p = jnp.concatenate([cols, pad_idx])
    vals_p = jnp.concatenate([vals, jnp.zeros((npad,), jnp.float32)])

    ue0 = user_emb_pre + user_emb
    ie0 = item_emb_pre + item_emb

    z_u1 = _spmm(rows_p, cols_p, vals_p, ie0)
    z_i1 = _spmm(cols_p, rows_p, vals_p, ue0)
    z_u2 = _spmm(rows_p, cols_p, vals_p, z_i1)
    z_i2 = _spmm(cols_p, rows_p, vals_p, z_u1)

    user_out = _combine(ue0, z_u1, z_u2, image_user_embeds, text_user_embeds)
    item_out = _combine(ie0, z_i1, z_i2, image_item_embeds, text_item_embeds)
    return (user_out, item_out)
